# baseline (speedup 1.0000x reference)
import functools

import jax
import jax.numpy as jnp
from jax import lax
from jax.experimental import pallas as pl
from jax.experimental.pallas import tpu as pltpu
from jax.experimental.pallas import tpu_sc as plsc

_B = 16384
_D = 128
_H = 1024
_NC = 2
_NS = 16
_NW = _NC * _NS
_CH = 128

_NCHUNK = 4
_ROWS = _B // _NCHUNK
_BPW = _ROWS // _NW
_NCH = _BPW // _CH

_BLK = 2048


def _gather_body(uidx_hbm, vidx_hbm, user_hbm, movie_hbm, h_out,
                 idx_v, rows_v, sem):
    wid = lax.axis_index("s") * _NC + lax.axis_index("c")
    base = wid * _BPW
    for col, idx_hbm, tbl in ((0, uidx_hbm, user_hbm),
                              (_D, vidx_hbm, movie_hbm)):
        pltpu.sync_copy(idx_hbm.at[pl.ds(base, _BPW)], idx_v)
        copies = [
            pltpu.async_copy(tbl.at[idx_v.at[pl.ds(c * _CH, _CH)]],
                             rows_v.at[pl.ds(c * _CH, _CH)], sem)
            for c in range(_NCH)
        ]
        for cp in copies:
            cp.wait()
        pltpu.sync_copy(rows_v, h_out.at[pl.ds(base, _BPW), pl.ds(col, _D)])


def _gather(uidx, vidx, user_emb, movie_emb):
    mesh = plsc.VectorSubcoreMesh(core_axis_name="c", subcore_axis_name="s",
                                  num_cores=_NC, num_subcores=_NS)
    return pl.kernel(
        _gather_body,
        out_type=jax.ShapeDtypeStruct((_ROWS, 2 * _D), jnp.float32),
        mesh=mesh,
        scratch_types=[
            pltpu.VMEM((_BPW,), jnp.int32),
            pltpu.VMEM((_BPW, _D), jnp.float32),
            pltpu.SemaphoreType.DMA,
        ],
    )(uidx, vidx, user_emb, movie_emb)


def _mlp_body(h_ref, w1_ref, b1_ref, w2_ref, b2_ref, acc_ref, out_ref):
    del acc_ref
    hin = jnp.maximum(h_ref[...].astype(jnp.bfloat16), 0)
    h = jnp.dot(hin, w1_ref[...], preferred_element_type=jnp.float32)
    h = jnp.maximum(h + b1_ref[...], 0.0).astype(jnp.bfloat16)
    logit = jax.lax.dot_general(w2_ref[...], h,
                                (((1,), (1,)), ((), ())),
                                preferred_element_type=jnp.float32)
    logit = logit + b2_ref[0, 0]
    out_ref[...] = 5.5 * jax.nn.sigmoid(-logit)


def _mlp(chunk, H, w1, b1, w2, b2, acc):
    grid = (_ROWS // _BLK,)
    off = chunk * (_ROWS // _BLK)
    return pl.pallas_call(
        _mlp_body,
        grid=grid,
        in_specs=[
            pl.BlockSpec((_BLK, 2 * _D), lambda i: (i, 0)),
            pl.BlockSpec((2 * _D, _H), lambda i: (0, 0)),
            pl.BlockSpec((1, _H), lambda i: (0, 0)),
            pl.BlockSpec((1, _H), lambda i: (0, 0)),
            pl.BlockSpec((1, 1), lambda i: (0, 0)),
            pl.BlockSpec(memory_space=pl.ANY),
        ],
        out_specs=pl.BlockSpec((1, _BLK), lambda i: (0, off + i)),
        out_shape=jax.ShapeDtypeStruct((1, _B), jnp.float32),
        input_output_aliases={5: 0},
    )(H, w1, b1, w2, b2, acc)


def kernel(x, user_emb, movie_emb, W1, b1, W2, b2):
    uidx = x[:, 0]
    vidx = x[:, 1]
    w1 = W1.astype(jnp.bfloat16)
    b1r = b1.reshape(1, _H)
    w2 = W2.reshape(1, _H).astype(jnp.bfloat16)
    b2r = b2.reshape(1, 1)
    out = jnp.zeros((1, _B), jnp.float32)
    for c in range(_NCHUNK):
        Hc = _gather(uidx[c * _ROWS:(c + 1) * _ROWS],
                     vidx[c * _ROWS:(c + 1) * _ROWS], user_emb, movie_emb)
        out = _mlp(c, Hc, w1, b1r, w2, b2r, out)
    return out.reshape(_B)


# speedup vs baseline: 1.1484x; 1.1484x over previous
import functools

import jax
import jax.numpy as jnp
from jax import lax
from jax.experimental import pallas as pl
from jax.experimental.pallas import tpu as pltpu
from jax.experimental.pallas import tpu_sc as plsc

_B = 16384
_D = 128
_H = 1024
_NC = 2
_NS = 16
_NW = _NC * _NS
_CH = 128

_NCHUNK = 2
_ROWS = _B // _NCHUNK
_BPW = _ROWS // _NW
_NCH = _BPW // _CH

_BLK = 2048


def _gather_body(chunk, uidx_hbm, vidx_hbm, user_hbm, movie_hbm, h_out,
                 idx_v, rows_v, sem):
    wid = lax.axis_index("s") * _NC + lax.axis_index("c")
    base = wid * _BPW
    for col, idx_hbm, tbl in ((0, uidx_hbm, user_hbm),
                              (_D, vidx_hbm, movie_hbm)):
        pltpu.sync_copy(idx_hbm.at[pl.ds(chunk * _ROWS + base, _BPW)], idx_v)
        copies = [
            pltpu.async_copy(tbl.at[idx_v.at[pl.ds(c * _CH, _CH)]],
                             rows_v.at[pl.ds(c * _CH, _CH)], sem)
            for c in range(_NCH)
        ]
        for cp in copies:
            cp.wait()
        pltpu.sync_copy(rows_v, h_out.at[pl.ds(base, _BPW), pl.ds(col, _D)])


def _gather(chunk, uidx, vidx, user_emb, movie_emb):
    mesh = plsc.VectorSubcoreMesh(core_axis_name="c", subcore_axis_name="s",
                                  num_cores=_NC, num_subcores=_NS)
    return pl.kernel(
        functools.partial(_gather_body, chunk),
        out_type=jax.ShapeDtypeStruct((_ROWS, 2 * _D), jnp.float32),
        mesh=mesh,
        scratch_types=[
            pltpu.VMEM((_BPW,), jnp.int32),
            pltpu.VMEM((_BPW, _D), jnp.float32),
            pltpu.SemaphoreType.DMA,
        ],
    )(uidx, vidx, user_emb, movie_emb)


def _mlp_body(h_ref, w1_ref, b1_ref, w2_ref, b2_ref, acc_ref, out_ref):
    del acc_ref
    hin = jnp.maximum(h_ref[...].astype(jnp.bfloat16), 0)
    h = jnp.dot(hin, w1_ref[...], preferred_element_type=jnp.float32)
    h = jnp.maximum(h + b1_ref[...], 0.0).astype(jnp.bfloat16)
    logit = jax.lax.dot_general(w2_ref[...], h,
                                (((1,), (1,)), ((), ())),
                                preferred_element_type=jnp.float32)
    logit = logit + b2_ref[0, 0]
    out_ref[...] = 5.5 * jax.nn.sigmoid(-logit)


def _mlp(chunk, H, w1, b1, w2, b2, acc):
    grid = (_ROWS // _BLK,)
    off = chunk * (_ROWS // _BLK)
    return pl.pallas_call(
        _mlp_body,
        grid=grid,
        in_specs=[
            pl.BlockSpec((_BLK, 2 * _D), lambda i: (i, 0)),
            pl.BlockSpec((2 * _D, _H), lambda i: (0, 0)),
            pl.BlockSpec((1, _H), lambda i: (0, 0)),
            pl.BlockSpec((1, _H), lambda i: (0, 0)),
            pl.BlockSpec((1, 1), lambda i: (0, 0)),
            pl.BlockSpec(memory_space=pl.ANY),
        ],
        out_specs=pl.BlockSpec((1, _BLK), lambda i: (0, off + i)),
        out_shape=jax.ShapeDtypeStruct((1, _B), jnp.float32),
        input_output_aliases={5: 0},
    )(H, w1, b1, w2, b2, acc)


def kernel(x, user_emb, movie_emb, W1, b1, W2, b2):
    uidx = x[:, 0]
    vidx = x[:, 1]
    w1 = W1.astype(jnp.bfloat16)
    b1r = b1.reshape(1, _H)
    w2 = W2.reshape(1, _H).astype(jnp.bfloat16)
    b2r = b2.reshape(1, 1)
    out = jnp.zeros((1, _B), jnp.float32)
    for c in range(_NCHUNK):
        Hc = _gather(c, uidx, vidx, user_emb, movie_emb)
        out = _mlp(c, Hc, w1, b1r, w2, b2r, out)
    return out.reshape(_B)


# speedup vs baseline: 1.1509x; 1.0021x over previous
import functools

import jax
import jax.numpy as jnp
from jax import lax
from jax.experimental import pallas as pl
from jax.experimental.pallas import tpu as pltpu
from jax.experimental.pallas import tpu_sc as plsc

_B = 16384
_D = 128
_H = 1024
_NC = 2
_NS = 16
_NW = _NC * _NS
_CH = 128

_NCHUNK = 2
_ROWS = _B // _NCHUNK
_BPW = _ROWS // _NW
_NCH = _BPW // _CH

_BLK = 2048


def _gather_body(chunk, uidx_hbm, vidx_hbm, user_hbm, movie_hbm, h_out,
                 idx_v, rows_v, sem):
    wid = lax.axis_index("s") * _NC + lax.axis_index("c")
    base = wid * _BPW
    for col, idx_hbm, tbl in ((0, uidx_hbm, user_hbm),
                              (_D, vidx_hbm, movie_hbm)):
        pltpu.sync_copy(idx_hbm.at[pl.ds(chunk * _ROWS + base, _BPW)], idx_v)
        copies = [
            pltpu.async_copy(tbl.at[idx_v.at[pl.ds(c * _CH, _CH)]],
                             rows_v.at[pl.ds(c * _CH, _CH)], sem)
            for c in range(_NCH)
        ]
        for cp in copies:
            cp.wait()
        pltpu.sync_copy(rows_v, h_out.at[pl.ds(base, _BPW), pl.ds(col, _D)])


def _gather(chunk, uidx, vidx, user_emb, movie_emb):
    mesh = plsc.VectorSubcoreMesh(core_axis_name="c", subcore_axis_name="s",
                                  num_cores=_NC, num_subcores=_NS)
    return pl.kernel(
        functools.partial(_gather_body, chunk),
        out_type=jax.ShapeDtypeStruct((_ROWS, 2 * _D), jnp.float32),
        mesh=mesh,
        scratch_types=[
            pltpu.VMEM((_BPW,), jnp.int32),
            pltpu.VMEM((_BPW, _D), jnp.float32),
            pltpu.SemaphoreType.DMA,
        ],
    )(uidx, vidx, user_emb, movie_emb)


def _mlp_body(h_ref, w1_ref, b1_ref, w2_ref, b2_ref, acc_ref, out_ref):
    del acc_ref
    hin = jnp.maximum(h_ref[...].astype(jnp.bfloat16), 0)
    h = jnp.dot(hin, w1_ref[...], preferred_element_type=jnp.float32)
    h = jnp.maximum(h + b1_ref[...], 0.0).astype(jnp.bfloat16)
    logit = jax.lax.dot_general(w2_ref[...], h,
                                (((1,), (1,)), ((), ())),
                                preferred_element_type=jnp.float32)
    logit = logit + b2_ref[0, 0]
    out_ref[...] = 5.5 * jax.nn.sigmoid(-logit)


def _mlp(chunk, H, w1, b1, w2, b2, acc):
    grid = (_ROWS // _BLK,)
    off = chunk * (_ROWS // _BLK)
    return pl.pallas_call(
        _mlp_body,
        grid=grid,
        in_specs=[
            pl.BlockSpec((_BLK, 2 * _D), lambda i: (i, 0)),
            pl.BlockSpec((2 * _D, _H), lambda i: (0, 0)),
            pl.BlockSpec((1, _H), lambda i: (0, 0)),
            pl.BlockSpec((1, _H), lambda i: (0, 0)),
            pl.BlockSpec((1, 1), lambda i: (0, 0)),
            pl.BlockSpec(memory_space=pl.ANY),
        ],
        out_specs=pl.BlockSpec((1, _BLK), lambda i: (0, off + i)),
        out_shape=jax.ShapeDtypeStruct((1, _B), jnp.float32),
        input_output_aliases={5: 0},
        compiler_params=pltpu.CompilerParams(
            dimension_semantics=("parallel",)),
    )(H, w1, b1, w2, b2, acc)


def kernel(x, user_emb, movie_emb, W1, b1, W2, b2):
    uidx = x[:, 0]
    vidx = x[:, 1]
    w1 = W1.astype(jnp.bfloat16)
    b1r = b1.reshape(1, _H)
    w2 = W2.reshape(1, _H).astype(jnp.bfloat16)
    b2r = b2.reshape(1, 1)
    out = jnp.zeros((1, _B), jnp.float32)
    for c in range(_NCHUNK):
        Hc = _gather(c, uidx, vidx, user_emb, movie_emb)
        out = _mlp(c, Hc, w1, b1r, w2, b2r, out)
    return out.reshape(_B)


# speedup vs baseline: 1.1539x; 1.0026x over previous
import functools

import jax
import jax.numpy as jnp
from jax import lax
from jax.experimental import pallas as pl
from jax.experimental.pallas import tpu as pltpu
from jax.experimental.pallas import tpu_sc as plsc

_B = 16384
_D = 128
_H = 1024
_NC = 2
_NS = 16
_NW = _NC * _NS
_CH = 128

_NCHUNK = 2
_ROWS = _B // _NCHUNK
_BPW = _ROWS // _NW
_NCH = _BPW // _CH

_BLK = 2048


def _gather_body(chunk, uidx_hbm, vidx_hbm, user_hbm, movie_hbm, h_out,
                 idx_v, rows_v, sem):
    wid = lax.axis_index("s") * _NC + lax.axis_index("c")
    base = wid * _BPW
    for col, idx_hbm, tbl in ((0, uidx_hbm, user_hbm),
                              (_D, vidx_hbm, movie_hbm)):
        pltpu.sync_copy(idx_hbm.at[pl.ds(chunk * _ROWS + base, _BPW)], idx_v)
        copies = [
            pltpu.async_copy(tbl.at[idx_v.at[pl.ds(c * _CH, _CH)]],
                             rows_v.at[pl.ds(c * _CH, _CH)], sem)
            for c in range(_NCH)
        ]
        for cp in copies:
            cp.wait()
        pltpu.sync_copy(rows_v, h_out.at[pl.ds(base, _BPW), pl.ds(col, _D)])


def _gather(chunk, uidx, vidx, user_emb, movie_emb):
    mesh = plsc.VectorSubcoreMesh(core_axis_name="c", subcore_axis_name="s",
                                  num_cores=_NC, num_subcores=_NS)
    return pl.kernel(
        functools.partial(_gather_body, chunk),
        out_type=jax.ShapeDtypeStruct((_ROWS, 2 * _D), jnp.float32),
        mesh=mesh,
        scratch_types=[
            pltpu.VMEM((_BPW,), jnp.int32),
            pltpu.VMEM((_BPW, _D), jnp.float32),
            pltpu.SemaphoreType.DMA,
        ],
    )(uidx, vidx, user_emb, movie_emb)


def _mlp_body(h_ref, w1_ref, b1_ref, w2_ref, b2_ref, *rest):
    out_ref = rest[-1]
    hin = jnp.maximum(h_ref[...].astype(jnp.bfloat16), 0)
    h = jnp.dot(hin, w1_ref[...], preferred_element_type=jnp.float32)
    h = jnp.maximum(h + b1_ref[...], 0.0).astype(jnp.bfloat16)
    logit = jax.lax.dot_general(w2_ref[...], h,
                                (((1,), (1,)), ((), ())),
                                preferred_element_type=jnp.float32)
    logit = logit + b2_ref[0, 0]
    out_ref[...] = 5.5 * jax.nn.sigmoid(-logit)


def _mlp(chunk, H, w1, b1, w2, b2, acc):
    grid = (_ROWS // _BLK,)
    off = chunk * (_ROWS // _BLK)
    in_specs = [
        pl.BlockSpec((_BLK, 2 * _D), lambda i: (i, 0)),
        pl.BlockSpec((2 * _D, _H), lambda i: (0, 0)),
        pl.BlockSpec((1, _H), lambda i: (0, 0)),
        pl.BlockSpec((1, _H), lambda i: (0, 0)),
        pl.BlockSpec((1, 1), lambda i: (0, 0)),
    ]
    args = [H, w1, b1, w2, b2]
    aliases = {}
    if acc is not None:
        in_specs.append(pl.BlockSpec(memory_space=pl.ANY))
        args.append(acc)
        aliases = {5: 0}
    return pl.pallas_call(
        _mlp_body,
        grid=grid,
        in_specs=in_specs,
        out_specs=pl.BlockSpec((1, _BLK), lambda i: (0, off + i)),
        out_shape=jax.ShapeDtypeStruct((1, _B), jnp.float32),
        input_output_aliases=aliases,
        compiler_params=pltpu.CompilerParams(
            dimension_semantics=("parallel",)),
    )(*args)


def kernel(x, user_emb, movie_emb, W1, b1, W2, b2):
    uidx = x[:, 0]
    vidx = x[:, 1]
    w1 = W1.astype(jnp.bfloat16)
    b1r = b1.reshape(1, _H)
    w2 = W2.reshape(1, _H).astype(jnp.bfloat16)
    b2r = b2.reshape(1, 1)
    out = None
    for c in range(_NCHUNK):
        Hc = _gather(c, uidx, vidx, user_emb, movie_emb)
        out = _mlp(c, Hc, w1, b1r, w2, b2r, out)
    return out.reshape(_B)
